# baseline (speedup 1.0000x reference)
import functools

import jax
import jax.numpy as jnp
from jax import lax
from jax.experimental import pallas as pl
from jax.experimental.pallas import tpu as pltpu
from jax.experimental.pallas import tpu_sc as plsc


def kernel(hidden_states, cls_index):
    B, S, H = hidden_states.shape
    half = B // 2
    nout = 2 * B
    table = hidden_states.reshape(B * S, H)

    mesh = plsc.ScalarSubcoreMesh(axis_name="c", num_cores=2)

    @functools.partial(
        pl.kernel,
        out_type=jax.ShapeDtypeStruct((nout, H), jnp.float32),
        mesh=mesh,
        scratch_types=[
            pltpu.SMEM((B,), jnp.int32),
            pltpu.SemaphoreType.DMA,
        ],
    )
    def gather_rows(table_hbm, cls_hbm, out_hbm, cls_s, sem):
        cid = lax.axis_index("c")
        pltpu.sync_copy(cls_hbm, cls_s)
        copies = []
        for k in range(B):
            row = min(k, half - 1) + half * cid
            copies.append(
                pltpu.make_async_copy(
                    table_hbm.at[pl.ds(row * S + cls_s[k], 1)],
                    out_hbm.at[pl.ds(cid * B + k, 1)],
                    sem,
                )
            )
        for c in copies:
            c.start()
        for c in copies:
            c.wait()

    return gather_rows(table, cls_index)


# speedup vs baseline: 1.0493x; 1.0493x over previous
import functools

import jax
import jax.numpy as jnp
from jax.experimental import pallas as pl
from jax.experimental.pallas import tpu as pltpu
from jax.experimental.pallas import tpu_sc as plsc


def kernel(hidden_states, cls_index):
    B, S, H = hidden_states.shape
    half = B // 2
    nout = 2 * B
    table = hidden_states.reshape(B * S, H)

    mesh = plsc.ScalarSubcoreMesh(axis_name="c", num_cores=1)

    @functools.partial(
        pl.kernel,
        out_type=jax.ShapeDtypeStruct((nout, H), jnp.float32),
        mesh=mesh,
        scratch_types=[
            pltpu.SMEM((B,), jnp.int32),
            pltpu.SemaphoreType.DMA,
        ],
    )
    def gather_rows(table_hbm, cls_hbm, out_hbm, cls_s, sem):
        pltpu.sync_copy(cls_hbm, cls_s)

        def issue(b, carry):
            j = b & (B - 1)
            row = jnp.minimum(j, half - 1) + half * (b >> (B.bit_length() - 1))
            pltpu.make_async_copy(
                table_hbm.at[pl.ds(row * S + cls_s[j], 1)],
                out_hbm.at[pl.ds(b, 1)],
                sem,
            ).start()
            return carry

        jax.lax.fori_loop(0, nout, issue, 0)

        def drain(b, carry):
            pltpu.make_async_copy(
                table_hbm.at[pl.ds(0, 1)], out_hbm.at[pl.ds(b, 1)], sem
            ).wait()
            return carry

        jax.lax.fori_loop(0, nout, drain, 0)

    return gather_rows(table, cls_index)
